# baseline (speedup 1.0000x reference)
import functools

import jax
import jax.numpy as jnp
from jax import lax
from jax.experimental import pallas as pl
from jax.experimental.pallas import tpu as pltpu
from jax.experimental.pallas import tpu_sc as plsc

_B = 16384
_D = 32
_NW = 32
_V = 1000000
_WPC = 5
_CW = _WPC * 128
_NCH = 49
_SPANW = 245
_T31W = 7595
_T31CH = 43
_SPEC0 = 999680
_SPECW = 256
_K = 48
_BP = _B + 8


def _hit_lists(u):
    w = u >> 7
    l = u & 127
    t = jnp.minimum(w // _SPANW, _NW - 1)
    bw = jnp.where(t < _NW - 1, t * _SPANW, _T31W)
    rel = w - bw
    spec = w >= 7810
    c = jnp.where(spec, _T31CH, rel // _WPC)
    col = jnp.where(spec,
                    jnp.where(u >= 999936, u - 999616, u - _SPEC0),
                    (rel % _WPC) * 128 + l)
    key = t * 64 + c
    order = jnp.argsort(key)
    ks = key[order]
    first = jnp.searchsorted(ks, ks, side="left")
    rank = jnp.clip(jnp.arange(_B) - first, 0, _K - 1)
    slot = (t[order] * _NCH + c[order]) * _K + rank
    nslots = _NW * _NCH * _K
    bs = jnp.full((nslots,), _B, jnp.int32).at[slot].set(
        order.astype(jnp.int32))
    cols = jnp.zeros((nslots,), jnp.int32).at[slot].set(
        col[order].astype(jnp.int32))
    return bs, cols


def _sc_gather(ub, ucol, ib, icol, t_um, t_im, t_ug, t_ig,
               z_um, z_im, z_ug, z_ig):
    mesh = plsc.VectorSubcoreMesh(core_axis_name="c", subcore_axis_name="s")

    @functools.partial(
        pl.kernel,
        out_type=[jax.ShapeDtypeStruct((_BP, 128), jnp.float32)] * 4,
        mesh=mesh,
        compiler_params=pltpu.CompilerParams(use_tc_tiling_on_sc=True,
                                             needs_layout_passes=False),
        scratch_types=[
            pltpu.VMEM((_K,), jnp.int32),
            pltpu.VMEM((_K,), jnp.int32),
            pltpu.VMEM((_K,), jnp.int32),
            pltpu.VMEM((_K,), jnp.int32),
            pltpu.VMEM((_D, _CW), jnp.float32),
            pltpu.VMEM((_D, _CW), jnp.float32),
            pltpu.VMEM((_D, _CW), jnp.float32),
            pltpu.VMEM((_D, _CW), jnp.float32),
            pltpu.VMEM((_K, 128), jnp.float32),
            pltpu.VMEM((_K, 128), jnp.float32),
            pltpu.VMEM((_K, 128), jnp.float32),
            pltpu.VMEM((_K, 128), jnp.float32),
            pltpu.SemaphoreType.DMA,
            pltpu.SemaphoreType.DMA,
        ],
    )
    def k(ub_h, ucol_h, ib_h, icol_h, um_h, im_h, ug_h, ig_h,
          z_um_h, z_im_h, z_ug_h, z_ig_h,
          o_um, o_im, o_ug, o_ig,
          ubv, ucolv, ibv, icolv,
          bum, bim, bug, big_, sum_, sim, sug, sig, gsem, wsem):
        wid = lax.axis_index("s") * 2 + lax.axis_index("c")
        span0 = pl.multiple_of(
            jnp.where(wid < _NW - 1, wid * _SPANW * 128, _T31W * 128), 128)
        limit = jnp.where(wid == _NW - 1, _T31CH, _NCH)

        iota = jnp.arange(16, dtype=jnp.int32)

        def extract(colv, bufs, stags):
            for g in range(_K // 16):
                colvec = colv[pl.ds(g * 16, 16)]
                svec = iota + g * 16
                for f in range(_D):
                    cf = jnp.full((16,), f, jnp.int32)
                    for buf, stag in zip(bufs, stags):
                        v = plsc.load_gather(buf, [cf, colvec])
                        plsc.store_scatter(stag, [svec, cf], v)

        def do_chunk(c, width, start, tails):
            cps = [
                pltpu.async_copy(um_h.at[:, pl.ds(start, width)],
                                 bum.at[:, pl.ds(0, width)], gsem),
                pltpu.async_copy(im_h.at[:, pl.ds(start, width)],
                                 bim.at[:, pl.ds(0, width)], gsem),
                pltpu.async_copy(ug_h.at[:, pl.ds(start, width)],
                                 bug.at[:, pl.ds(0, width)], gsem),
                pltpu.async_copy(ig_h.at[:, pl.ds(start, width)],
                                 big_.at[:, pl.ds(0, width)], gsem),
            ]
            if tails:
                for z, buf in zip((z_um_h, z_im_h, z_ug_h, z_ig_h),
                                  (bum, bim, bug, big_)):
                    cps.append(pltpu.async_copy(
                        z, buf.at[:, pl.ds(_SPECW, 128)], gsem))
            hb = pl.ds((wid * _NCH + c) * _K, _K)
            pltpu.sync_copy(ub_h.at[hb], ubv)
            pltpu.sync_copy(ucol_h.at[hb], ucolv)
            pltpu.sync_copy(ib_h.at[hb], ibv)
            pltpu.sync_copy(icol_h.at[hb], icolv)
            for cp in cps:
                cp.wait()

            extract(ucolv, (bum, bug), (sum_, sug))
            extract(icolv, (bim, big_), (sim, sig))
            sc = [
                pltpu.async_copy(sum_, o_um.at[ubv], wsem),
                pltpu.async_copy(sim, o_im.at[ibv], wsem),
                pltpu.async_copy(sug, o_ug.at[ubv], wsem),
                pltpu.async_copy(sig, o_ig.at[ibv], wsem),
            ]
            for s_ in sc:
                s_.wait()

        def body(c, _):
            @pl.when(c < limit)
            def _():
                do_chunk(c, _CW, span0 + c * _CW, False)

            @pl.when((wid == _NW - 1) & (c == _T31CH))
            def _():
                do_chunk(c, _SPECW, _SPEC0, True)

            return 0

        lax.fori_loop(0, _NCH, body, 0)

    return k(ub, ucol, ib, icol, t_um, t_im, t_ug, t_ig,
             z_um, z_im, z_ug, z_ig)


def _dense_body(ue_ref, ie_ref, ug_ref, ig_ref, w1u_ref, w1i_ref, b1_ref,
                w2_ref, b2_ref, wh_ref, wg_ref, bo_ref, o_ref):
    ue = ue_ref[...][:, :_D]
    ie = ie_ref[...][:, :_D]
    ug = ug_ref[...][:, :_D]
    ig = ig_ref[...][:, :_D]
    h1 = jnp.dot(ue, w1u_ref[...], preferred_element_type=jnp.float32)
    h1 = h1 + jnp.dot(ie, w1i_ref[...], preferred_element_type=jnp.float32)
    h1 = jnp.maximum(h1 + b1_ref[...], 0.0)
    h2 = jnp.dot(h1, w2_ref[...], preferred_element_type=jnp.float32)
    h2 = jnp.maximum(h2 + b2_ref[...], 0.0)
    gmf = ug * ig
    logit = jnp.dot(h2, wh_ref[...], preferred_element_type=jnp.float32)
    logit = logit + jnp.dot(gmf, wg_ref[...], preferred_element_type=jnp.float32)
    o_ref[...] = logit + bo_ref[...]


def _tc_dense(ue, ie, ug, ig, w1u, w1i, b1, w2, b2, wh, wg, bo):
    bb = 2048
    grid = _B // bb
    row_spec = pl.BlockSpec((bb, 128), lambda i: (i, 0))

    def w_spec(shape):
        return pl.BlockSpec(shape, lambda i: (0,) * len(shape))

    return pl.pallas_call(
        _dense_body,
        grid=(grid,),
        in_specs=[
            row_spec, row_spec, row_spec, row_spec,
            w_spec((_D, 32)), w_spec((_D, 32)), w_spec((1, 32)),
            w_spec((32, 16)), w_spec((1, 16)),
            w_spec((16, 1)), w_spec((_D, 1)), w_spec((1, 1)),
        ],
        out_specs=pl.BlockSpec((bb, 1), lambda i: (i, 0)),
        out_shape=jax.ShapeDtypeStruct((_B, 1), jnp.float32),
    )(ue, ie, ug, ig, w1u, w1i, b1, w2, b2, wh, wg, bo)


def kernel(userID, itemID, user_emb_mlp, item_emb_mlp, user_emb_gmf,
           item_emb_gmf, W1, b1, W2, b2, Wout, bout):
    uid = userID.astype(jnp.int32)
    iid = itemID.astype(jnp.int32)
    ub, ucol = _hit_lists(uid)
    ib, icol = _hit_lists(iid)
    tails = [t.T[:, 999872:] for t in (user_emb_mlp, item_emb_mlp,
                                       user_emb_gmf, item_emb_gmf)]
    ue, iem, ug, ig = _sc_gather(ub, ucol, ib, icol,
                                 user_emb_mlp.T, item_emb_mlp.T,
                                 user_emb_gmf.T, item_emb_gmf.T, *tails)
    out = _tc_dense(ue, iem, ug, ig,
                    W1[:_D], W1[_D:], b1.reshape(1, -1),
                    W2, b2.reshape(1, -1),
                    Wout[:16], Wout[16:], bout.reshape(1, 1))
    return out.reshape(-1)


# speedup vs baseline: 2.3393x; 2.3393x over previous
import functools

import jax
import jax.numpy as jnp
from jax import lax
from jax.experimental import pallas as pl
from jax.experimental.pallas import tpu as pltpu
from jax.experimental.pallas import tpu_sc as plsc

_B = 16384
_D = 32
_DP = 4 * _D
_NC = 2
_NS = 16
_NW = _NC * _NS
_BPW = _B // _NW
_CH = 128
_NCH = _BPW // _CH


def _sc_gather(qu, qi, t_um, t_im, t_ug, t_ig):
    mesh = plsc.VectorSubcoreMesh(core_axis_name="c", subcore_axis_name="s")

    @functools.partial(
        pl.kernel,
        out_type=[jax.ShapeDtypeStruct((_B, _DP), jnp.float32)] * 4,
        mesh=mesh,
        compiler_params=pltpu.CompilerParams(use_tc_tiling_on_sc=True),
        scratch_types=[
            pltpu.VMEM((_BPW,), jnp.int32),
            pltpu.VMEM((_BPW,), jnp.int32),
            pltpu.VMEM((_CH, _DP), jnp.float32),
            pltpu.VMEM((_CH, _DP), jnp.float32),
            pltpu.VMEM((_CH, _DP), jnp.float32),
            pltpu.VMEM((_CH, _DP), jnp.float32),
            pltpu.SemaphoreType.DMA,
            pltpu.SemaphoreType.DMA,
        ],
    )
    def k(qu_hbm, qi_hbm, um_hbm, im_hbm, ug_hbm, ig_hbm,
          o_um, o_im, o_ug, o_ig,
          qu_v, qi_v, um_v, im_v, ug_v, ig_v, gsem, wsem):
        wid = lax.axis_index("s") * _NC + lax.axis_index("c")
        base = wid * _BPW
        pltpu.sync_copy(qu_hbm.at[pl.ds(base, _BPW)], qu_v)
        pltpu.sync_copy(qi_hbm.at[pl.ds(base, _BPW)], qi_v)
        for c in range(_NCH):
            s = pl.ds(c * _CH, _CH)
            o = pl.ds(base + c * _CH, _CH)
            gs = [
                pltpu.async_copy(um_hbm.at[qu_v.at[s]], um_v, gsem),
                pltpu.async_copy(im_hbm.at[qi_v.at[s]], im_v, gsem),
                pltpu.async_copy(ug_hbm.at[qu_v.at[s]], ug_v, gsem),
                pltpu.async_copy(ig_hbm.at[qi_v.at[s]], ig_v, gsem),
            ]
            for g in gs:
                g.wait()
            ws = [
                pltpu.async_copy(um_v, o_um.at[o], wsem),
                pltpu.async_copy(im_v, o_im.at[o], wsem),
                pltpu.async_copy(ug_v, o_ug.at[o], wsem),
                pltpu.async_copy(ig_v, o_ig.at[o], wsem),
            ]
            for w in ws:
                w.wait()

    return k(qu, qi, t_um, t_im, t_ug, t_ig)


def _sel4(x, r):
    acc = jnp.where(r == 0, x[:, 0:_D], 0.0)
    for p in range(1, 4):
        acc = acc + jnp.where(r == p, x[:, p * _D:(p + 1) * _D], 0.0)
    return acc


def _dense_body(um_ref, im_ref, ug_ref, ig_ref, ru_ref, ri_ref,
                w1u_ref, w1i_ref, b1_ref, w2_ref, b2_ref,
                wh_ref, wg_ref, bo_ref, o_ref):
    ru = ru_ref[...]
    ri = ri_ref[...]
    ue = _sel4(um_ref[...], ru)
    ie = _sel4(im_ref[...], ri)
    ug = _sel4(ug_ref[...], ru)
    ig = _sel4(ig_ref[...], ri)
    h1 = jnp.dot(ue, w1u_ref[...], preferred_element_type=jnp.float32)
    h1 = h1 + jnp.dot(ie, w1i_ref[...], preferred_element_type=jnp.float32)
    h1 = jnp.maximum(h1 + b1_ref[...], 0.0)
    h2 = jnp.dot(h1, w2_ref[...], preferred_element_type=jnp.float32)
    h2 = jnp.maximum(h2 + b2_ref[...], 0.0)
    gmf = ug * ig
    logit = jnp.dot(h2, wh_ref[...], preferred_element_type=jnp.float32)
    logit = logit + jnp.dot(gmf, wg_ref[...], preferred_element_type=jnp.float32)
    o_ref[...] = logit + bo_ref[...]


def _tc_dense(um, im, ug, ig, ru, ri, w1u, w1i, b1, w2, b2, wh, wg, bo):
    bb = 2048
    grid = _B // bb
    row_spec = pl.BlockSpec((bb, _DP), lambda i: (i, 0))
    r_spec = pl.BlockSpec((bb, 1), lambda i: (i, 0))

    def w_spec(shape):
        return pl.BlockSpec(shape, lambda i: (0,) * len(shape))

    return pl.pallas_call(
        _dense_body,
        grid=(grid,),
        in_specs=[
            row_spec, row_spec, row_spec, row_spec, r_spec, r_spec,
            w_spec((_D, 32)), w_spec((_D, 32)), w_spec((1, 32)),
            w_spec((32, 16)), w_spec((1, 16)),
            w_spec((16, 1)), w_spec((_D, 1)), w_spec((1, 1)),
        ],
        out_specs=pl.BlockSpec((bb, 1), lambda i: (i, 0)),
        out_shape=jax.ShapeDtypeStruct((_B, 1), jnp.float32),
    )(um, im, ug, ig, ru, ri, w1u, w1i, b1, w2, b2, wh, wg, bo)


def kernel(userID, itemID, user_emb_mlp, item_emb_mlp, user_emb_gmf,
           item_emb_gmf, W1, b1, W2, b2, Wout, bout):
    uid = userID.astype(jnp.int32)
    iid = itemID.astype(jnp.int32)
    qu = uid >> 2
    qi = iid >> 2
    ru = (uid & 3).reshape(-1, 1)
    ri = (iid & 3).reshape(-1, 1)
    np_rows = user_emb_mlp.shape[0] // 4
    t_um = user_emb_mlp.reshape(np_rows, _DP)
    t_im = item_emb_mlp.reshape(np_rows, _DP)
    t_ug = user_emb_gmf.reshape(np_rows, _DP)
    t_ig = item_emb_gmf.reshape(np_rows, _DP)
    um, imr, ugr, igr = _sc_gather(qu, qi, t_um, t_im, t_ug, t_ig)
    out = _tc_dense(um, imr, ugr, igr, ru, ri,
                    W1[:_D], W1[_D:], b1.reshape(1, -1),
                    W2, b2.reshape(1, -1),
                    Wout[:16], Wout[16:], bout.reshape(1, 1))
    return out.reshape(-1)
